# baseline (speedup 1.0000x reference)
import jax
import jax.numpy as jnp
from jax import lax
from jax.experimental import pallas as pl
from jax.experimental.pallas import tpu as pltpu, tpu_sc as plsc

D = 1024
B, S = 4, 2048
N = B * S
NC, NS = 2, 16
NW = NC * NS
PER_W = N // NW
CHUNK = 32
NCHUNK = PER_W // CHUNK
LANES = 16
VPR = D // LANES


def _emb_body(ids_hbm, tok_hbm, pos_hbm, out_hbm,
              idx_v, rows_v, pos_v, sem_g, sem_p, sem_o):
    wid = lax.axis_index("s") * NC + lax.axis_index("c")
    base = wid * PER_W
    s_base = lax.rem(base, S)
    for k in range(NCHUNK):
        tok_off = base + k * CHUNK
        pos_off = s_base + k * CHUNK
        pltpu.sync_copy(ids_hbm.at[pl.ds(tok_off, CHUNK)], idx_v)
        cp_g = pltpu.async_copy(tok_hbm.at[idx_v], rows_v, sem_g)
        cp_p = pltpu.async_copy(pos_hbm.at[pl.ds(pos_off, CHUNK)], pos_v, sem_p)
        cp_g.wait()
        cp_p.wait()

        def row_body(r, carry):
            for c in range(VPR):
                sl = pl.ds(c * LANES, LANES)
                plsc.addupdate(rows_v.at[r, sl], pos_v[r, sl])
            return carry

        lax.fori_loop(0, CHUNK, row_body, 0)
        pltpu.async_copy(rows_v, out_hbm.at[pl.ds(tok_off, CHUNK)], sem_o).wait()


def kernel(input_ids, token_table, pos_table):
    ids_flat = input_ids.reshape(N).astype(jnp.int32)
    mesh = plsc.VectorSubcoreMesh(core_axis_name="c", subcore_axis_name="s")
    out = pl.kernel(
        _emb_body,
        out_type=jax.ShapeDtypeStruct((N, D), jnp.float32),
        mesh=mesh,
        scratch_types=[
            pltpu.VMEM((CHUNK,), jnp.int32),
            pltpu.VMEM((CHUNK, D), jnp.float32),
            pltpu.VMEM((CHUNK, D), jnp.float32),
            pltpu.SemaphoreType.DMA,
            pltpu.SemaphoreType.DMA,
            pltpu.SemaphoreType.DMA,
        ],
    )(ids_flat, token_table, pos_table)
    return out.reshape(B, S, D)


# speedup vs baseline: 1.1889x; 1.1889x over previous
import jax
import jax.numpy as jnp
from jax import lax
from jax.experimental import pallas as pl
from jax.experimental.pallas import tpu as pltpu, tpu_sc as plsc

D = 1024
B, S = 4, 2048
N = B * S
NC, NS = 2, 16
NW = NC * NS
SPW = S // NW
CHUNK = 16
NCHUNK = (SPW // CHUNK) * B
LANES = 16
VPR = D // LANES


def _emb_body(ids_hbm, tok_hbm, pos_hbm, out_hbm,
              idx_v, pos_v, rows0, rows1, sem_g0, sem_g1, sem_o0, sem_o1):
    wid = lax.axis_index("s") * NC + lax.axis_index("c")
    s_base = wid * SPW
    pltpu.sync_copy(pos_hbm.at[pl.ds(s_base, SPW)], pos_v)
    for b in range(B):
        pltpu.sync_copy(ids_hbm.at[pl.ds(b * S + s_base, SPW)],
                        idx_v.at[pl.ds(b * SPW, SPW)])

    rows = (rows0, rows1)
    sem_g = (sem_g0, sem_g1)
    sem_o = (sem_o0, sem_o1)
    spc = SPW // CHUNK

    def flat_off(k):
        b, sub = divmod(k, spc)
        return b * S + s_base + sub * CHUNK

    def start_gather(k):
        p = k % 2
        return pltpu.async_copy(
            tok_hbm.at[idx_v.at[pl.ds(k * CHUNK, CHUNK)]], rows[p], sem_g[p])

    start_gather(0)
    for k in range(NCHUNK):
        p = k % 2
        pltpu.make_async_copy(
            tok_hbm.at[idx_v.at[pl.ds(k * CHUNK, CHUNK)]], rows[p],
            sem_g[p]).wait()
        if k + 1 < NCHUNK:
            if k >= 1:
                pltpu.make_async_copy(
                    rows[1 - p], out_hbm.at[pl.ds(flat_off(k - 1), CHUNK)],
                    sem_o[1 - p]).wait()
            start_gather(k + 1)

        sub = k % spc
        pos_row0 = sub * CHUNK

        def row_body(r, carry):
            for c in range(VPR):
                sl = pl.ds(c * LANES, LANES)
                plsc.addupdate(rows[p].at[r, sl], pos_v[pos_row0 + r, sl])
            return carry

        lax.fori_loop(0, CHUNK, row_body, 0)
        pltpu.async_copy(rows[p], out_hbm.at[pl.ds(flat_off(k), CHUNK)],
                         sem_o[p])
    for k in (NCHUNK - 2, NCHUNK - 1):
        p = k % 2
        pltpu.make_async_copy(
            rows[p], out_hbm.at[pl.ds(flat_off(k), CHUNK)], sem_o[p]).wait()


def kernel(input_ids, token_table, pos_table):
    ids_flat = input_ids.reshape(N).astype(jnp.int32)
    mesh = plsc.VectorSubcoreMesh(core_axis_name="c", subcore_axis_name="s")
    out = pl.kernel(
        _emb_body,
        out_type=jax.ShapeDtypeStruct((N, D), jnp.float32),
        mesh=mesh,
        scratch_types=[
            pltpu.VMEM((B * SPW,), jnp.int32),
            pltpu.VMEM((SPW, D), jnp.float32),
            pltpu.VMEM((CHUNK, D), jnp.float32),
            pltpu.VMEM((CHUNK, D), jnp.float32),
            pltpu.SemaphoreType.DMA,
            pltpu.SemaphoreType.DMA,
            pltpu.SemaphoreType.DMA,
            pltpu.SemaphoreType.DMA,
        ],
    )(ids_flat, token_table, pos_table)
    return out.reshape(B, S, D)


# speedup vs baseline: 1.3634x; 1.1467x over previous
import jax
import jax.numpy as jnp
from jax import lax
from jax.experimental import pallas as pl
from jax.experimental.pallas import tpu as pltpu, tpu_sc as plsc

D = 1024
B, S = 4, 2048
N = B * S
NC, NS = 2, 16
NW = NC * NS
SPW = S // NW
CHUNK = 16
NBUF = 3
NCHUNK = (SPW // CHUNK) * B
LANES = 16
VPR = D // LANES


def _emb_body(ids_hbm, tok_hbm, pos_hbm, out_hbm,
              idx_v, pos_v, rows0, rows1, rows2,
              sem_p, sem_g0, sem_g1, sem_g2, sem_o0, sem_o1, sem_o2):
    wid = lax.axis_index("s") * NC + lax.axis_index("c")
    s_base = wid * SPW
    cp_pos = pltpu.async_copy(pos_hbm.at[pl.ds(s_base, SPW)], pos_v, sem_p)
    for b in range(B):
        pltpu.sync_copy(ids_hbm.at[pl.ds(b * S + s_base, SPW)],
                        idx_v.at[pl.ds(b * SPW, SPW)])

    rows = (rows0, rows1, rows2)
    sem_g = (sem_g0, sem_g1, sem_g2)
    sem_o = (sem_o0, sem_o1, sem_o2)
    spc = SPW // CHUNK

    def flat_off(k):
        b, sub = divmod(k, spc)
        return b * S + s_base + sub * CHUNK

    def gather_cp(k):
        p = k % NBUF
        return pltpu.make_async_copy(
            tok_hbm.at[idx_v.at[pl.ds(k * CHUNK, CHUNK)]], rows[p], sem_g[p])

    def out_cp(k):
        p = k % NBUF
        return pltpu.make_async_copy(
            rows[p], out_hbm.at[pl.ds(flat_off(k), CHUNK)], sem_o[p])

    gather_cp(0).start()
    gather_cp(1).start()
    cp_pos.wait()
    for k in range(NCHUNK):
        p = k % NBUF
        gather_cp(k).wait()

        sub = k % spc
        pos_row0 = sub * CHUNK

        def row_body(r, carry):
            for c in range(VPR):
                sl = pl.ds(c * LANES, LANES)
                plsc.addupdate(rows[p].at[r, sl], pos_v[pos_row0 + r, sl])
            return carry

        lax.fori_loop(0, CHUNK, row_body, 0)
        out_cp(k).start()
        if k + 2 < NCHUNK:
            if k >= 1:
                out_cp(k - 1).wait()
            gather_cp(k + 2).start()
    for k in (NCHUNK - 3, NCHUNK - 2, NCHUNK - 1):
        out_cp(k).wait()


def kernel(input_ids, token_table, pos_table):
    ids_flat = input_ids.reshape(N).astype(jnp.int32)
    mesh = plsc.VectorSubcoreMesh(core_axis_name="c", subcore_axis_name="s")
    out = pl.kernel(
        _emb_body,
        out_type=jax.ShapeDtypeStruct((N, D), jnp.float32),
        mesh=mesh,
        scratch_types=[
            pltpu.VMEM((B * SPW,), jnp.int32),
            pltpu.VMEM((SPW, D), jnp.float32),
            pltpu.VMEM((CHUNK, D), jnp.float32),
            pltpu.VMEM((CHUNK, D), jnp.float32),
            pltpu.VMEM((CHUNK, D), jnp.float32),
            pltpu.SemaphoreType.DMA,
            pltpu.SemaphoreType.DMA,
            pltpu.SemaphoreType.DMA,
            pltpu.SemaphoreType.DMA,
            pltpu.SemaphoreType.DMA,
            pltpu.SemaphoreType.DMA,
            pltpu.SemaphoreType.DMA,
        ],
    )(ids_flat, token_table, pos_table)
    return out.reshape(B, S, D)
